# baseline (speedup 1.0000x reference)
import functools

import jax
import jax.numpy as jnp
from jax import lax
from jax.experimental import pallas as pl
from jax.experimental.pallas import tpu as pltpu
from jax.experimental.pallas import tpu_sc as plsc

NC, NS = 2, 16
NW = NC * NS

V = 1000000
E = 64
B, H = 4096, 200
VT2 = 3906
BPW = 123
RROWS = 500032
U2 = (H * 16)
UPW = U2 // NW


def _wid():
    return lax.axis_index("s") * NC + lax.axis_index("c")


def _mesh():
    return plsc.VectorSubcoreMesh(
        core_axis_name="c", subcore_axis_name="s", num_cores=NC, num_subcores=NS
    )


@functools.partial(
    pl.kernel,
    out_type=jax.ShapeDtypeStruct((RROWS, 128), jnp.float32),
    mesh=_mesh(),
    compiler_params=pltpu.CompilerParams(
        use_tc_tiling_on_sc=True, needs_layout_passes=False
    ),
    scratch_types=[
        pltpu.VMEM((2, 64, 256), jnp.float32),
        pltpu.VMEM((2, 128, 128), jnp.float32),
        pltpu.VMEM((64, 64), jnp.float32),
        pltpu.SemaphoreType.DMA((2,)),
        pltpu.SemaphoreType.DMA((2,)),
    ],
)
def _transpose_sc(tT, r_out, src_v, dst_v, tail_v, rsem, wsem):
    wid = _wid()
    lo = wid * BPW
    hi = jnp.minimum(lo + BPW, VT2)

    iot = lax.iota(jnp.int32, 16)
    rowv = []
    colb = []
    for kc in range(16):
        c = iot + kc * 16
        rowv.append(lax.shift_right_logical(c, 1))
        colb.append((c & 1) * 64)

    def read(tc, p):
        pltpu.async_copy(tT.at[:, pl.ds(tc * 256, 256)], src_v.at[p], rsem.at[p])

    def wait_read(p):
        pltpu.make_async_copy(tT.at[:, pl.ds(0, 256)], src_v.at[p], rsem.at[p]).wait()

    def write(tc, p):
        pltpu.async_copy(dst_v.at[p], r_out.at[pl.ds(tc * 128, 128)], wsem.at[p])

    def wait_write(p):
        pltpu.make_async_copy(
            r_out.at[pl.ds(0, 128)], dst_v.at[p], wsem.at[p]
        ).wait()

    read(lo, 0)
    read(lo + 1, 1)

    @pl.loop(lo, hi)
    def _(tc):
        i = tc - lo
        p = i & 1
        wait_read(p)

        @pl.when(i >= 2)
        def _():
            wait_write(p)

        @plsc.parallel_loop(0, 64, unroll=8)
        def _(e):
            for kc in range(16):
                plsc.store_scatter(
                    dst_v.at[p],
                    [rowv[kc], colb[kc] + e],
                    src_v[p, e, pl.ds(kc * 16, 16)],
                )

        @pl.when(tc + 2 < hi)
        def _():
            read(tc + 2, p)

        write(tc, p)

    n = hi - lo
    wait_write(n & 1)
    wait_write((n + 1) & 1)

    @pl.when(wid == NW - 1)
    def _():
        pltpu.sync_copy(tT.at[:, pl.ds(VT2 * 256, 64)], tail_v)

        @plsc.parallel_loop(0, 64, unroll=8)
        def _(e):
            for kc in range(4):
                plsc.store_scatter(
                    dst_v.at[0], [rowv[kc], colb[kc] + e], tail_v[e, pl.ds(kc * 16, 16)]
                )

        pltpu.sync_copy(dst_v.at[0, pl.ds(0, 32)], r_out.at[pl.ds(VT2 * 128, 32)])


@functools.partial(
    pl.kernel,
    out_type=jax.ShapeDtypeStruct((H, 8, 32, 8, 128), jnp.float32),
    mesh=_mesh(),
    compiler_params=pltpu.CompilerParams(
        use_tc_tiling_on_sc=False, needs_layout_passes=False
    ),
    scratch_types=[
        pltpu.VMEM((2, 256), jnp.int32),
        pltpu.VMEM((2, 256, 64), jnp.float32),
        pltpu.VMEM((2, 8, 2, 8, 128), jnp.float32),
        pltpu.SemaphoreType.DMA((2,)),
        pltpu.SemaphoreType.DMA((2,)),
    ],
)
def _gather_sc(xT, r_in, out5, xbuf, gbuf, dst_v, gsem, osem):
    wid = _wid()
    base = wid * UPW

    iot = lax.iota(jnp.int32, 16)
    r0 = []
    r1 = []
    for kc in range(4):
        e = iot + kc * 16
        r0.append(lax.shift_right_logical(e, 3))
        r1.append(e & 7)

    def xload(u):
        h = u // 16
        b2 = (u % 16) * 256
        pltpu.sync_copy(xT.at[h, pl.ds(b2, 256)], xbuf.at[u & 1])

    def gather(u):
        pltpu.async_copy(r_in.at[xbuf.at[u & 1]], gbuf.at[u & 1], gsem.at[u & 1])

    def wait_gather(p):
        pltpu.make_async_copy(r_in.at[pl.ds(0, 256)], gbuf.at[p], gsem.at[p]).wait()

    def owrite(u):
        h = u // 16
        btc = (u % 16) * 2
        pltpu.async_copy(
            dst_v.at[u & 1], out5.at[h, :, pl.ds(btc, 2)], osem.at[u & 1]
        )

    def wait_owrite(p):
        pltpu.make_async_copy(
            out5.at[0, :, pl.ds(0, 2)], dst_v.at[p], osem.at[p]
        ).wait()

    xload(base)
    gather(base)
    xload(base + 1)

    @pl.loop(base, base + UPW)
    def _(u):
        i = u - base
        p = i & 1

        @pl.when(i + 1 < UPW)
        def _():
            gather(u + 1)

        wait_gather(p)

        @pl.when(i >= 2)
        def _():
            wait_owrite(p)

        @plsc.parallel_loop(0, 256, unroll=8)
        def _(cc):
            d1 = jnp.full((16,), lax.shift_right_logical(cc, 7), jnp.int32)
            d3 = jnp.full((16,), cc & 127, jnp.int32)
            for kc in range(4):
                plsc.store_scatter(
                    dst_v.at[p],
                    [r0[kc], d1, r1[kc], d3],
                    gbuf[p, cc, pl.ds(kc * 16, 16)],
                )

        @pl.when(i + 2 < UPW)
        def _():
            xload(u + 2)

        owrite(u)

    wait_owrite(UPW & 1)
    wait_owrite((UPW + 1) & 1)


def kernel(x, table):
    r = _transpose_sc(table.T)
    out5 = _gather_sc(x.T.astype(jnp.int32), r.reshape(1000064, 64))
    return out5.transpose(2, 4, 0, 1, 3).reshape(B, H, E)


# speedup vs baseline: 3.9816x; 3.9816x over previous
import functools

import jax
import jax.numpy as jnp
from jax import lax
from jax.experimental import pallas as pl
from jax.experimental.pallas import tpu as pltpu
from jax.experimental.pallas import tpu_sc as plsc

NC, NS = 2, 16
NW = NC * NS

V = 1000000
E = 64
B, H = 4096, 200
VT2 = 3906
BPW = 123
RROWS = 500032
U2 = (H * 16)
UPW = U2 // NW


def _wid():
    return lax.axis_index("s") * NC + lax.axis_index("c")


def _mesh():
    return plsc.VectorSubcoreMesh(
        core_axis_name="c", subcore_axis_name="s", num_cores=NC, num_subcores=NS
    )


@functools.partial(
    pl.kernel,
    out_type=jax.ShapeDtypeStruct((RROWS, 128), jnp.float32),
    mesh=_mesh(),
    compiler_params=pltpu.CompilerParams(
        use_tc_tiling_on_sc=True, needs_layout_passes=False
    ),
    scratch_types=[
        pltpu.VMEM((2, 64, 256), jnp.float32),
        pltpu.VMEM((2, 128, 128), jnp.float32),
        pltpu.VMEM((64, 64), jnp.float32),
        pltpu.SemaphoreType.DMA((2,)),
        pltpu.SemaphoreType.DMA((2,)),
    ],
)
def _transpose_sc(tT, r_out, src_v, dst_v, tail_v, rsem, wsem):
    wid = _wid()
    lo = wid * BPW
    hi = jnp.minimum(lo + BPW, VT2)

    iot = lax.iota(jnp.int32, 16)

    def read(tc, p):
        pltpu.async_copy(tT.at[:, pl.ds(tc * 256, 256)], src_v.at[p], rsem.at[p])

    def wait_read(p):
        pltpu.make_async_copy(tT.at[:, pl.ds(0, 256)], src_v.at[p], rsem.at[p]).wait()

    def write(tc, p):
        pltpu.async_copy(dst_v.at[p], r_out.at[pl.ds(tc * 128, 128)], wsem.at[p])

    def wait_write(p):
        pltpu.make_async_copy(
            r_out.at[pl.ds(0, 128)], dst_v.at[p], wsem.at[p]
        ).wait()

    read(lo, 0)
    read(lo + 1, 1)

    @pl.loop(lo, hi)
    def _(tc):
        i = tc - lo
        p = i & 1
        wait_read(p)

        @pl.when(i >= 2)
        def _():
            wait_write(p)

        @plsc.parallel_loop(0, 64, unroll=8)
        def _(e0):
            ev = (e0 + iot) & 63
            for kc in range(16):
                cv = iot + kc * 16
                vals = plsc.load_gather(src_v.at[p], [ev, cv])
                plsc.store_scatter(
                    dst_v.at[p],
                    [lax.shift_right_logical(cv, 1), (cv & 1) * 64 + ev],
                    vals,
                )

        @pl.when(tc + 2 < hi)
        def _():
            read(tc + 2, p)

        write(tc, p)

    n = hi - lo
    wait_write(n & 1)
    wait_write((n + 1) & 1)

    @pl.when(wid == NW - 1)
    def _():
        pltpu.sync_copy(tT.at[:, pl.ds(VT2 * 256, 64)], tail_v)

        @plsc.parallel_loop(0, 64, unroll=8)
        def _(e0):
            ev = (e0 + iot) & 63
            for kc in range(4):
                cv = iot + kc * 16
                vals = plsc.load_gather(tail_v, [ev, cv])
                plsc.store_scatter(
                    dst_v.at[0],
                    [lax.shift_right_logical(cv, 1), (cv & 1) * 64 + ev],
                    vals,
                )

        pltpu.sync_copy(dst_v.at[0, pl.ds(0, 32)], r_out.at[pl.ds(VT2 * 128, 32)])


@functools.partial(
    pl.kernel,
    out_type=jax.ShapeDtypeStruct((H, 8, 32, 8, 128), jnp.float32),
    mesh=_mesh(),
    compiler_params=pltpu.CompilerParams(
        use_tc_tiling_on_sc=False, needs_layout_passes=False
    ),
    scratch_types=[
        pltpu.VMEM((2, 256), jnp.int32),
        pltpu.VMEM((2, 256, 64), jnp.float32),
        pltpu.VMEM((2, 8, 2, 8, 128), jnp.float32),
        pltpu.SemaphoreType.DMA((2,)),
        pltpu.SemaphoreType.DMA((2,)),
    ],
)
def _gather_sc(xT, r_in, out5, xbuf, gbuf, dst_v, gsem, osem):
    wid = _wid()
    base = wid * UPW

    iot = lax.iota(jnp.int32, 16)

    def xload(u):
        h = u // 16
        b2 = (u % 16) * 256
        pltpu.sync_copy(xT.at[h, pl.ds(b2, 256)], xbuf.at[u & 1])

    def gather(u):
        pltpu.async_copy(r_in.at[xbuf.at[u & 1]], gbuf.at[u & 1], gsem.at[u & 1])

    def wait_gather(p):
        pltpu.make_async_copy(r_in.at[pl.ds(0, 256)], gbuf.at[p], gsem.at[p]).wait()

    def owrite(u):
        h = u // 16
        btc = (u % 16) * 2
        pltpu.async_copy(
            dst_v.at[u & 1], out5.at[h, :, pl.ds(btc, 2)], osem.at[u & 1]
        )

    def wait_owrite(p):
        pltpu.make_async_copy(
            out5.at[0, :, pl.ds(0, 2)], dst_v.at[p], osem.at[p]
        ).wait()

    xload(base)
    gather(base)
    xload(base + 1)

    @pl.loop(base, base + UPW)
    def _(u):
        i = u - base
        p = i & 1

        @pl.when(i + 1 < UPW)
        def _():
            gather(u + 1)

        wait_gather(p)

        @pl.when(i >= 2)
        def _():
            wait_owrite(p)

        @plsc.parallel_loop(0, 64, unroll=8)
        def _(e0):
            ev = (e0 + iot) & 63
            d0 = lax.shift_right_logical(ev, 3)
            d2 = ev & 7
            for kc in range(16):
                cv = iot + kc * 16
                vals = plsc.load_gather(gbuf.at[p], [cv, ev])
                plsc.store_scatter(
                    dst_v.at[p],
                    [d0, lax.shift_right_logical(cv, 7), d2, cv & 127],
                    vals,
                )

        @pl.when(i + 2 < UPW)
        def _():
            xload(u + 2)

        owrite(u)

    wait_owrite(UPW & 1)
    wait_owrite((UPW + 1) & 1)


def kernel(x, table):
    r = _transpose_sc(table.T)
    out5 = _gather_sc(x.T.astype(jnp.int32), r.reshape(1000064, 64))
    return out5.transpose(2, 4, 0, 1, 3).reshape(B, H, E)
